# baseline (speedup 1.0000x reference)
import functools

import jax
import jax.numpy as jnp
from jax import lax
from jax.experimental import pallas as pl
from jax.experimental.pallas import tpu as pltpu
from jax.experimental.pallas import tpu_sc as plsc

VOCAB = 1000000
DIM = 64
BATCH = 4096
HIST = 50
N = BATCH * HIST

_info = plsc.get_sparse_core_info()
_NC = _info.num_cores
_NS = _info.num_subcores
NW = _NC * _NS
B_PER_W = N // NW
CHUNK = 800
NCHUNK = B_PER_W // CHUNK

_mesh = plsc.VectorSubcoreMesh(core_axis_name="c", subcore_axis_name="s")


@functools.partial(
    pl.kernel,
    mesh=_mesh,
    compiler_params=pltpu.CompilerParams(use_tc_tiling_on_sc=False),
    out_type=jax.ShapeDtypeStruct((N, DIM), jnp.float32),
    scratch_types=[
        pltpu.VMEM((B_PER_W,), jnp.int32),
        pltpu.VMEM((2, CHUNK, DIM), jnp.float32),
        pltpu.SemaphoreType.DMA,
        pltpu.SemaphoreType.DMA,
    ],
)
def _emb_lookup(ids_hbm, table_hbm, out_hbm, idx_v, rows_v, sem0, sem1):
    wid = lax.axis_index("s") * _NC + lax.axis_index("c")
    base = wid * B_PER_W
    pltpu.sync_copy(ids_hbm.at[pl.ds(base, B_PER_W)], idx_v)
    sems = (sem0, sem1)
    copies = [None, None]
    copies[0] = pltpu.async_copy(
        table_hbm.at[idx_v.at[pl.ds(0, CHUNK)]], rows_v.at[0], sems[0]
    )
    for g in range(NCHUNK):
        nxt = (g + 1) % 2
        if g + 1 < NCHUNK:
            copies[nxt] = pltpu.async_copy(
                table_hbm.at[idx_v.at[pl.ds((g + 1) * CHUNK, CHUNK)]],
                rows_v.at[nxt],
                sems[nxt],
            )
        copies[g % 2].wait()
        pltpu.sync_copy(
            rows_v.at[g % 2], out_hbm.at[pl.ds(base + g * CHUNK, CHUNK)]
        )


def kernel(input_ids, table):
    ids = input_ids.reshape(-1).astype(jnp.int32)
    out = _emb_lookup(ids, table)
    return out.reshape(BATCH, HIST, DIM)


# speedup vs baseline: 1.5083x; 1.5083x over previous
import functools

import jax
import jax.numpy as jnp
from jax import lax
from jax.experimental import pallas as pl
from jax.experimental.pallas import tpu as pltpu
from jax.experimental.pallas import tpu_sc as plsc

VOCAB = 1000000
DIM = 64
BATCH = 4096
HIST = 50
N = BATCH * HIST

_info = plsc.get_sparse_core_info()
_NC = _info.num_cores
_NS = _info.num_subcores
NW = _NC * _NS
B_PER_W = BATCH // NW
ROWS_PER_W = B_PER_W * HIST
CB = 8
CR = CB * HIST
NCHUNK = B_PER_W // CB

_mesh = plsc.VectorSubcoreMesh(core_axis_name="c", subcore_axis_name="s")


@functools.partial(
    pl.kernel,
    mesh=_mesh,
    out_type=jax.ShapeDtypeStruct((BATCH, HIST, DIM), jnp.float32),
    scratch_types=[
        pltpu.VMEM((ROWS_PER_W,), jnp.int32),
        pltpu.VMEM((2, CB, HIST, DIM), jnp.float32),
        pltpu.SemaphoreType.DMA,
        pltpu.SemaphoreType.DMA,
        pltpu.SemaphoreType.DMA,
        pltpu.SemaphoreType.DMA,
    ],
)
def _emb_lookup(ids_hbm, table_hbm, out_hbm, idx_v, buf, g0, g1, o0, o1):
    wid = lax.axis_index("s") * _NC + lax.axis_index("c")
    base_row = wid * ROWS_PER_W
    base_batch = wid * B_PER_W
    pltpu.sync_copy(ids_hbm.at[pl.ds(base_row, ROWS_PER_W)], idx_v)
    gsems = (g0, g1)
    osems = (o0, o1)

    def issue_chunk(g, slot):
        def body(t, carry):
            v = idx_v[pl.ds(g * CR + t * 16, 16)]
            for u in range(16):
                j = t * 16 + u
                pltpu.async_copy(
                    table_hbm.at[v[u]],
                    buf.at[slot, j // HIST, j % HIST],
                    gsems[slot],
                )
            return carry
        lax.fori_loop(0, CR // 16, body, 0)

    def drain_chunk(slot):
        pltpu.make_async_copy(
            out_hbm.at[pl.ds(0, CB)], buf.at[slot], gsems[slot]
        ).wait()

    def write_chunk(g, slot):
        return pltpu.async_copy(
            buf.at[slot], out_hbm.at[pl.ds(base_batch + g * CB, CB)], osems[slot]
        )

    def wait_write(slot):
        pltpu.make_async_copy(
            buf.at[slot], out_hbm.at[pl.ds(base_batch, CB)], osems[slot]
        ).wait()

    issue_chunk(0, 0)
    drain_chunk(0)
    write_chunk(0, 0)
    for g in range(1, NCHUNK):
        slot = g % 2
        if g >= 2:
            wait_write(slot)
        issue_chunk(g, slot)
        drain_chunk(slot)
        write_chunk(g, slot)
    wait_write(0)
    wait_write(1)


def kernel(input_ids, table):
    ids = input_ids.reshape(-1).astype(jnp.int32)
    return _emb_lookup(ids, table)
